# baseline (speedup 1.0000x reference)
import functools

import jax
import jax.numpy as jnp
from jax.experimental import pallas as pl
from jax.experimental.pallas import tpu as pltpu

_B, _S, _DM = 2, 2048, 2048
_DB = 512
_NQK, _NV, _NKNOW = 4096, 4096, 8192
_KQK, _KV, _KKNOW = 32, 32, 8
_T = _B * _S


def _norm_body(e_ref, o_ref):
    e = e_ref[...]
    n = jnp.sqrt(jnp.sum(e * e, axis=-1, keepdims=True))
    o_ref[...] = e / (n + 1e-08)


def _normalize(emb):
    n, d = emb.shape
    blk = 1024
    return pl.pallas_call(
        _norm_body,
        grid=(n // blk,),
        in_specs=[pl.BlockSpec((blk, d), lambda i: (i, 0))],
        out_specs=pl.BlockSpec((blk, d), lambda i: (i, 0)),
        out_shape=jax.ShapeDtypeStruct((n, d), jnp.float32),
    )(emb)


def _tau_body(x_ref, w_ref, b_ref, o_ref):
    o_ref[...] = (
        jnp.dot(x_ref[...], w_ref[...], preferred_element_type=jnp.float32)
        + b_ref[...]
    )


def _tau(x2d, W4, b4):
    blk = 512
    return pl.pallas_call(
        _tau_body,
        grid=(_T // blk,),
        in_specs=[
            pl.BlockSpec((blk, _DM), lambda i: (i, 0)),
            pl.BlockSpec((_DM, 8), lambda i: (0, 0)),
            pl.BlockSpec((1, 8), lambda i: (0, 0)),
        ],
        out_specs=pl.BlockSpec((blk, 8), lambda i: (i, 0)),
        out_shape=jax.ShapeDtypeStruct((_T, 8), jnp.float32),
    )(x2d, W4, b4)


def _score_body(nsteps, n, tb, x_ref, w_ref, b_ref, e_ref,
                s_ref, m_ref, aux_ref, acc_ref):
    i = pl.program_id(0)
    h = (
        jnp.dot(x_ref[...], w_ref[...], preferred_element_type=jnp.float32)
        + b_ref[...]
    )
    s = jax.lax.dot_general(
        h, e_ref[...], (((1,), (1,)), ((), ())),
        preferred_element_type=jnp.float32,
    )
    s_ref[...] = s
    m_ref[...] = jnp.max(s.reshape(tb, n // 16, 16), axis=-1)
    rowmax = jnp.max(s, axis=-1, keepdims=True)
    p = jnp.exp(s - rowmax)
    rs = jnp.sum(p, axis=-1, keepdims=True)

    @pl.when(i == 0)
    def _():
        acc_ref[...] = jnp.zeros_like(acc_ref)

    acc_ref[...] += jnp.sum(p / rs, axis=0, keepdims=True)

    @pl.when(i == nsteps - 1)
    def _():
        u = acc_ref[...] / _T
        aux_ref[...] = jnp.sum((u - 1.0 / n) ** 2).reshape(1, 1)


def _scores(x2d, W, b, emb_n, n, tb):
    nsteps = _T // tb
    return pl.pallas_call(
        functools.partial(_score_body, nsteps, n, tb),
        grid=(nsteps,),
        in_specs=[
            pl.BlockSpec((tb, _DM), lambda i: (i, 0)),
            pl.BlockSpec((_DM, _DB), lambda i: (0, 0)),
            pl.BlockSpec((1, _DB), lambda i: (0, 0)),
            pl.BlockSpec((n, _DB), lambda i: (0, 0)),
        ],
        out_specs=[
            pl.BlockSpec((tb, n), lambda i: (i, 0)),
            pl.BlockSpec((tb, n // 16), lambda i: (i, 0)),
            pl.BlockSpec((1, 1), lambda i: (0, 0)),
        ],
        out_shape=[
            jax.ShapeDtypeStruct((_T, n), jnp.float32),
            jax.ShapeDtypeStruct((_T, n // 16), jnp.float32),
            jax.ShapeDtypeStruct((1, 1), jnp.float32),
        ],
        scratch_shapes=[pltpu.VMEM((1, n), jnp.float32)],
    )(x2d, W, b, emb_n)


def _gate_body(s_ref, tau_ref, g_ref):
    s = s_ref[...]
    tau = tau_ref[...]
    raw = s - tau
    gate = jnp.where(raw > 0, raw, 1e-08 * jnp.exp(raw))
    eg = jnp.exp(gate) - 1.0
    gsum = eg.sum(axis=-1, keepdims=True) + 1e-08
    gstr = jnp.tanh(eg.max(axis=-1, keepdims=True))
    g_ref[...] = eg / gsum * gstr


def _gate(topk_scores, tau, k):
    blk = 512
    return pl.pallas_call(
        _gate_body,
        grid=(_T // blk,),
        in_specs=[
            pl.BlockSpec((blk, k), lambda i: (i, 0)),
            pl.BlockSpec((blk, 1), lambda i: (i, 0)),
        ],
        out_specs=pl.BlockSpec((blk, k), lambda i: (i, 0)),
        out_shape=jax.ShapeDtypeStruct((_T, k), jnp.float32),
    )(topk_scores, tau)


def kernel(x, qk_emb, v_emb, know_emb, W_attn, b_attn, W_tau_attn, b_tau_attn,
           W_know, b_know, W_tau_know, b_tau_know):
    x2d = x.reshape(_T, _DM)
    qk_n = _normalize(qk_emb)
    v_n = _normalize(v_emb)
    know_n = _normalize(know_emb)

    W4 = jnp.concatenate([W_tau_attn, W_tau_know], axis=1)
    W4 = jnp.pad(W4, ((0, 0), (0, 4)))
    b4 = jnp.pad(jnp.concatenate([b_tau_attn, b_tau_know]), (0, 4)).reshape(1, 8)
    tau_all = _tau(x2d, W4, b4)

    def route(wslice, bslice, emb_n, n, tb, k, tau_col):
        s, m, aux = _scores(x2d, wslice, bslice, emb_n, n, tb)
        tv, ti = jax.lax.top_k(s, k)
        g = _gate(tv, tau_all[:, tau_col:tau_col + 1], k)
        return g.reshape(_B, _S, k), ti.reshape(_B, _S, k), aux[0, 0]

    g_Q, i_Q, aux_q = route(W_attn[:, 0:_DB], b_attn[0:_DB].reshape(1, _DB),
                            qk_n, _NQK, 128, _KQK, 0)
    g_K, i_K, _ = route(W_attn[:, _DB:2 * _DB], b_attn[_DB:2 * _DB].reshape(1, _DB),
                        qk_n, _NQK, 128, _KQK, 1)
    g_V, i_V, aux_v = route(W_attn[:, 2 * _DB:], b_attn[2 * _DB:].reshape(1, _DB),
                            v_n, _NV, 128, _KV, 2)
    g_know, i_know, aux_k = route(W_know, b_know.reshape(1, _DB),
                                  know_n, _NKNOW, 64, _KKNOW, 3)

    aux_attn = aux_q * _NQK * 3 + aux_v * _NV
    aux_know = aux_k * _NKNOW
    return (g_Q, i_Q, g_K, i_K, g_V, i_V, aux_attn, g_know, i_know, aux_know)


# speedup vs baseline: 8.6204x; 8.6204x over previous
import functools

import jax
import jax.numpy as jnp
from jax import lax
from jax.experimental import pallas as pl
from jax.experimental.pallas import tpu as pltpu
from jax.experimental.pallas import tpu_sc as plsc

_B, _S, _DM = 2, 2048, 2048
_DB = 512
_NQK, _NV, _NKNOW = 4096, 4096, 8192
_KQK, _KV, _KKNOW = 32, 32, 8
_T = _B * _S


def _norm_body(e_ref, o_ref):
    e = e_ref[...]
    n = jnp.sqrt(jnp.sum(e * e, axis=-1, keepdims=True))
    o_ref[...] = e / (n + 1e-08)


def _normalize(emb):
    n, d = emb.shape
    blk = 1024
    return pl.pallas_call(
        _norm_body,
        grid=(n // blk,),
        in_specs=[pl.BlockSpec((blk, d), lambda i: (i, 0))],
        out_specs=pl.BlockSpec((blk, d), lambda i: (i, 0)),
        out_shape=jax.ShapeDtypeStruct((n, d), jnp.float32),
    )(emb)


def _tau_body(x_ref, w_ref, b_ref, o_ref):
    o_ref[...] = (
        jnp.dot(x_ref[...], w_ref[...], preferred_element_type=jnp.float32)
        + b_ref[...]
    )


def _tau(x2d, W4, b4):
    blk = 512
    return pl.pallas_call(
        _tau_body,
        grid=(_T // blk,),
        in_specs=[
            pl.BlockSpec((blk, _DM), lambda i: (i, 0)),
            pl.BlockSpec((_DM, 8), lambda i: (0, 0)),
            pl.BlockSpec((1, 8), lambda i: (0, 0)),
        ],
        out_specs=pl.BlockSpec((blk, 8), lambda i: (i, 0)),
        out_shape=jax.ShapeDtypeStruct((_T, 8), jnp.float32),
    )(x2d, W4, b4)


def _score_body(nsteps, n, tb, x_ref, w_ref, b_ref, e_ref,
                s_ref, m_ref, aux_ref, acc_ref):
    i = pl.program_id(0)
    h = (
        jnp.dot(x_ref[...], w_ref[...], preferred_element_type=jnp.float32)
        + b_ref[...]
    )
    s = jax.lax.dot_general(
        h, e_ref[...], (((1,), (1,)), ((), ())),
        preferred_element_type=jnp.float32,
    )
    s_ref[...] = s
    m_ref[...] = jnp.max(s.reshape(tb, n // 16, 16), axis=-1)
    rowmax = jnp.max(s, axis=-1, keepdims=True)
    p = jnp.exp(s - rowmax)
    rs = jnp.sum(p, axis=-1, keepdims=True)

    @pl.when(i == 0)
    def _():
        acc_ref[...] = jnp.zeros_like(acc_ref)

    acc_ref[...] += jnp.sum(p / rs, axis=0, keepdims=True)

    @pl.when(i == nsteps - 1)
    def _():
        u = acc_ref[...] / _T
        aux_ref[...] = jnp.sum((u - 1.0 / n) ** 2).reshape(1, 1)


def _scores(x2d, W, b, emb_n, n, tb):
    nsteps = _T // tb
    return pl.pallas_call(
        functools.partial(_score_body, nsteps, n, tb),
        grid=(nsteps,),
        in_specs=[
            pl.BlockSpec((tb, _DM), lambda i: (i, 0)),
            pl.BlockSpec((_DM, _DB), lambda i: (0, 0)),
            pl.BlockSpec((1, _DB), lambda i: (0, 0)),
            pl.BlockSpec((n, _DB), lambda i: (0, 0)),
        ],
        out_specs=[
            pl.BlockSpec((tb, n), lambda i: (i, 0)),
            pl.BlockSpec((tb, n // 16), lambda i: (i, 0)),
            pl.BlockSpec((1, 1), lambda i: (0, 0)),
        ],
        out_shape=[
            jax.ShapeDtypeStruct((_T, n), jnp.float32),
            jax.ShapeDtypeStruct((_T, n // 16), jnp.float32),
            jax.ShapeDtypeStruct((1, 1), jnp.float32),
        ],
        scratch_shapes=[pltpu.VMEM((1, n), jnp.float32)],
    )(x2d, W, b, emb_n)


_NSC, _NSUB = 2, 16
_NW = _NSC * _NSUB


def _kvsort(p):
    v, i = p
    return plsc.sort_key_val(v, i)


def _kvrev(p):
    v, i = p
    return lax.rev(v, (0,)), lax.rev(i, (0,))


def _kvminmax(a, b):
    c = a[0] <= b[0]
    mn = (jnp.where(c, a[0], b[0]), jnp.where(c, a[1], b[1]))
    mx = (jnp.where(c, b[0], a[0]), jnp.where(c, b[1], a[1]))
    return mn, mx


def _merge_full_1(a, b):
    br = _kvrev(b)
    mn, mx = _kvminmax(a, br)
    return [_kvsort(mn), _kvsort(mx)]


def _merge_top_1(a, b):
    br = _kvrev(b)
    mn, _ = _kvminmax(a, br)
    return [_kvsort(mn)]


def _merge_top_2(A, B):
    brl = _kvrev(B[1])
    brh = _kvrev(B[0])
    x0, _ = _kvminmax(A[0], brl)
    x1, _ = _kvminmax(A[1], brh)
    u, w = _kvminmax(x0, x1)
    return [_kvsort(u), _kvsort(w)]


def _tree_topk(load_fn, n_leaves, r):
    def build(lo, hi):
        if hi - lo == 1:
            return [_kvsort(load_fn(lo))]
        mid = (lo + hi) // 2
        A = build(lo, mid)
        B = build(mid, hi)
        if r == 1:
            return _merge_top_1(A[0], B[0])
        if len(A) == 1:
            return _merge_full_1(A[0], B[0])
        return _merge_top_2(A, B)

    return build(0, n_leaves)


def _make_topk_sc(n, k, kpad, bt):
    m = n // 16
    nv = m // 16
    tw = _T // _NW
    nb = tw // bt
    r = 2 if k > 16 else 1

    mesh = plsc.VectorSubcoreMesh(core_axis_name="c", subcore_axis_name="s")

    @functools.partial(
        pl.kernel,
        mesh=mesh,
        out_type=[
            jax.ShapeDtypeStruct((_T * kpad,), jnp.float32),
            jax.ShapeDtypeStruct((_T * kpad,), jnp.int32),
        ],
        scratch_types=[
            pltpu.VMEM((bt * n,), jnp.float32),
            pltpu.VMEM((bt * m,), jnp.float32),
            pltpu.VMEM((bt * k + 16,), jnp.int32),
            pltpu.VMEM((bt * kpad,), jnp.float32),
            pltpu.VMEM((bt * kpad,), jnp.int32),
        ],
        compiler_params=pltpu.CompilerParams(needs_layout_passes=False),
    )
    def topk_sc(s_hbm, mx_hbm, ov_hbm, oi_hbm,
                rows_ref, mx_ref, cand_ref, outv_ref, outi_ref):
        wid = lax.axis_index("s") * _NSC + lax.axis_index("c")
        wbase = wid * tw
        ii = lax.iota(jnp.int32, 16)

        def batch_body(b, _):
            base = wbase + b * bt
            pltpu.sync_copy(mx_hbm.at[pl.ds(base * m, bt * m)], mx_ref)
            pltpu.sync_copy(s_hbm.at[pl.ds(base * n, bt * n)], rows_ref)

            def tok_body(t, _):
                off = t * m

                def load_mx(j):
                    v = -mx_ref[pl.ds(off + j * 16, 16)]
                    return v, v

                node = _tree_topk(load_mx, nv, r)
                tnode = node[1][0] if k > 16 else node[0][0]
                tval = -tnode[(k - 1) % 16]

                def scan_body(j, prev):
                    v = mx_ref[pl.ds(off + j * 16, 16)]
                    q = v >= tval
                    qi = q.astype(jnp.int32)
                    cs = plsc.cumsum(qi)
                    pos = prev + cs - 1
                    wm = q & (pos < k)
                    plsc.store_scatter(cand_ref, [t * k + pos],
                                       j * 16 + ii, mask=wm)
                    return prev + jnp.sum(qi)

                lax.fori_loop(0, nv, scan_body, jnp.int32(0))

                cvecs = [cand_ref[pl.ds(t * k + 16 * c, 16)]
                         for c in range((k + 15) // 16)]

                def load_cand(j):
                    cid = cvecs[j // 16][j % 16]
                    eidx = cid * 16 + ii
                    v = plsc.load_gather(rows_ref, [t * n + eidx])
                    return -v, eidx

                node = _tree_topk(load_cand, k, r)
                obase = t * kpad
                outv_ref[pl.ds(obase, 16)] = -node[0][0]
                outi_ref[pl.ds(obase, 16)] = node[0][1]
                if k > 16:
                    outv_ref[pl.ds(obase + 16, 16)] = -node[1][0]
                    outi_ref[pl.ds(obase + 16, 16)] = node[1][1]
                return 0

            lax.fori_loop(0, bt, tok_body, 0)

            pltpu.sync_copy(outv_ref, ov_hbm.at[pl.ds(base * kpad, bt * kpad)])
            pltpu.sync_copy(outi_ref, oi_hbm.at[pl.ds(base * kpad, bt * kpad)])
            return 0

        lax.fori_loop(0, nb, batch_body, 0)

    return topk_sc


def _topk_sc(scores, maxima, n, k):
    kpad = k if k % 16 == 0 else 16
    bt = 16 if n <= 4096 else 8
    sflat = scores.reshape(_T * n)
    mxflat = maxima.reshape(_T * (n // 16))
    ov, oi = _make_topk_sc(n, k, kpad, bt)(sflat, mxflat)
    tv = ov.reshape(_T, kpad)[:, :k]
    ti = oi.reshape(_T, kpad)[:, :k]
    return tv, ti


def _gate_body(s_ref, tau_ref, g_ref):
    s = s_ref[...]
    tau = tau_ref[...]
    raw = s - tau
    gate = jnp.where(raw > 0, raw, 1e-08 * jnp.exp(raw))
    eg = jnp.exp(gate) - 1.0
    gsum = eg.sum(axis=-1, keepdims=True) + 1e-08
    gstr = jnp.tanh(eg.max(axis=-1, keepdims=True))
    g_ref[...] = eg / gsum * gstr


def _gate(topk_scores, tau, k):
    blk = 512
    return pl.pallas_call(
        _gate_body,
        grid=(_T // blk,),
        in_specs=[
            pl.BlockSpec((blk, k), lambda i: (i, 0)),
            pl.BlockSpec((blk, 1), lambda i: (i, 0)),
        ],
        out_specs=pl.BlockSpec((blk, k), lambda i: (i, 0)),
        out_shape=jax.ShapeDtypeStruct((_T, k), jnp.float32),
    )(topk_scores, tau)


def kernel(x, qk_emb, v_emb, know_emb, W_attn, b_attn, W_tau_attn, b_tau_attn,
           W_know, b_know, W_tau_know, b_tau_know):
    x2d = x.reshape(_T, _DM)
    qk_n = _normalize(qk_emb)
    v_n = _normalize(v_emb)
    know_n = _normalize(know_emb)

    W4 = jnp.concatenate([W_tau_attn, W_tau_know], axis=1)
    W4 = jnp.pad(W4, ((0, 0), (0, 4)))
    b4 = jnp.pad(jnp.concatenate([b_tau_attn, b_tau_know]), (0, 4)).reshape(1, 8)
    tau_all = _tau(x2d, W4, b4)

    def route(wslice, bslice, emb_n, n, tb, k, tau_col):
        s, m, aux = _scores(x2d, wslice, bslice, emb_n, n, tb)
        tv, ti = _topk_sc(s, m, n, k)
        g = _gate(tv, tau_all[:, tau_col:tau_col + 1], k)
        return g.reshape(_B, _S, k), ti.reshape(_B, _S, k), aux[0, 0]

    g_Q, i_Q, aux_q = route(W_attn[:, 0:_DB], b_attn[0:_DB].reshape(1, _DB),
                            qk_n, _NQK, 128, _KQK, 0)
    g_K, i_K, _ = route(W_attn[:, _DB:2 * _DB], b_attn[_DB:2 * _DB].reshape(1, _DB),
                        qk_n, _NQK, 128, _KQK, 1)
    g_V, i_V, aux_v = route(W_attn[:, 2 * _DB:], b_attn[2 * _DB:].reshape(1, _DB),
                            v_n, _NV, 128, _KV, 2)
    g_know, i_know, aux_k = route(W_know, b_know.reshape(1, _DB),
                                  know_n, _NKNOW, 64, _KKNOW, 3)

    aux_attn = aux_q * _NQK * 3 + aux_v * _NV
    aux_know = aux_k * _NKNOW
    return (g_Q, i_Q, g_K, i_K, g_V, i_V, aux_attn, g_know, i_know, aux_know)


# speedup vs baseline: 11.3654x; 1.3184x over previous
import functools

import jax
import jax.numpy as jnp
from jax import lax
from jax.experimental import pallas as pl
from jax.experimental.pallas import tpu as pltpu
from jax.experimental.pallas import tpu_sc as plsc

_B, _S, _DM = 2, 2048, 2048
_DB = 512
_NQK, _NV, _NKNOW = 4096, 4096, 8192
_KQK, _KV, _KKNOW = 32, 32, 8
_T = _B * _S


def _norm_body(e_ref, o_ref):
    e = e_ref[...]
    n = jnp.sqrt(jnp.sum(e * e, axis=-1, keepdims=True))
    o_ref[...] = e / (n + 1e-08)


def _normalize(emb):
    n, d = emb.shape
    blk = 1024
    return pl.pallas_call(
        _norm_body,
        grid=(n // blk,),
        in_specs=[pl.BlockSpec((blk, d), lambda i: (i, 0))],
        out_specs=pl.BlockSpec((blk, d), lambda i: (i, 0)),
        out_shape=jax.ShapeDtypeStruct((n, d), jnp.float32),
    )(emb)


def _tau_body(x_ref, w_ref, b_ref, o_ref):
    o_ref[...] = (
        jnp.dot(x_ref[...], w_ref[...], preferred_element_type=jnp.float32)
        + b_ref[...]
    )


def _tau(x2d, W4, b4):
    blk = 512
    return pl.pallas_call(
        _tau_body,
        grid=(_T // blk,),
        in_specs=[
            pl.BlockSpec((blk, _DM), lambda i: (i, 0)),
            pl.BlockSpec((_DM, 8), lambda i: (0, 0)),
            pl.BlockSpec((1, 8), lambda i: (0, 0)),
        ],
        out_specs=pl.BlockSpec((blk, 8), lambda i: (i, 0)),
        out_shape=jax.ShapeDtypeStruct((_T, 8), jnp.float32),
    )(x2d, W4, b4)


def _score_body(nsteps, n, tb, x_ref, w_ref, b_ref, e_ref,
                s_ref, m_ref, aux_ref, acc_ref):
    i = pl.program_id(0)
    h = (
        jnp.dot(x_ref[...], w_ref[...], preferred_element_type=jnp.float32)
        + b_ref[...]
    )
    s = jax.lax.dot_general(
        h, e_ref[...], (((1,), (1,)), ((), ())),
        preferred_element_type=jnp.float32,
    )
    s_ref[...] = s
    m_ref[...] = jnp.max(s.reshape(tb, n // 16, 16), axis=-1)
    rowmax = jnp.max(s, axis=-1, keepdims=True)
    p = jnp.exp(s - rowmax)
    rs = jnp.sum(p, axis=-1, keepdims=True)

    @pl.when(i == 0)
    def _():
        acc_ref[...] = jnp.zeros_like(acc_ref)

    acc_ref[...] += jnp.sum(p / rs, axis=0, keepdims=True)

    @pl.when(i == nsteps - 1)
    def _():
        u = acc_ref[...] / _T
        aux_ref[...] = jnp.sum((u - 1.0 / n) ** 2).reshape(1, 1)


def _scores(x2d, W, b, emb_n, n, tb):
    nsteps = _T // tb
    return pl.pallas_call(
        functools.partial(_score_body, nsteps, n, tb),
        grid=(nsteps,),
        in_specs=[
            pl.BlockSpec((tb, _DM), lambda i: (i, 0)),
            pl.BlockSpec((_DM, _DB), lambda i: (0, 0)),
            pl.BlockSpec((1, _DB), lambda i: (0, 0)),
            pl.BlockSpec((n, _DB), lambda i: (0, 0)),
        ],
        out_specs=[
            pl.BlockSpec((tb, n), lambda i: (i, 0)),
            pl.BlockSpec((tb, n // 16), lambda i: (i, 0)),
            pl.BlockSpec((1, 1), lambda i: (0, 0)),
        ],
        out_shape=[
            jax.ShapeDtypeStruct((_T, n), jnp.float32),
            jax.ShapeDtypeStruct((_T, n // 16), jnp.float32),
            jax.ShapeDtypeStruct((1, 1), jnp.float32),
        ],
        scratch_shapes=[pltpu.VMEM((1, n), jnp.float32)],
    )(x2d, W, b, emb_n)


_NSC, _NSUB = 2, 16
_NW = _NSC * _NSUB


def _kvsort(p):
    v, i = p
    return plsc.sort_key_val(v, i)


def _kvrev(p):
    v, i = p
    return lax.rev(v, (0,)), lax.rev(i, (0,))


def _kvminmax(a, b):
    c = a[0] <= b[0]
    mn = (jnp.where(c, a[0], b[0]), jnp.where(c, a[1], b[1]))
    mx = (jnp.where(c, b[0], a[0]), jnp.where(c, b[1], a[1]))
    return mn, mx


def _merge_full_1(a, b):
    br = _kvrev(b)
    mn, mx = _kvminmax(a, br)
    return [_kvsort(mn), _kvsort(mx)]


def _merge_top_1(a, b):
    br = _kvrev(b)
    mn, _ = _kvminmax(a, br)
    return [_kvsort(mn)]


def _merge_top_2(A, B):
    brl = _kvrev(B[1])
    brh = _kvrev(B[0])
    x0, _ = _kvminmax(A[0], brl)
    x1, _ = _kvminmax(A[1], brh)
    u, w = _kvminmax(x0, x1)
    return [_kvsort(u), _kvsort(w)]


def _tree_topk(load_fn, n_leaves, r):
    def build(lo, hi):
        if hi - lo == 1:
            return [_kvsort(load_fn(lo))]
        mid = (lo + hi) // 2
        A = build(lo, mid)
        B = build(mid, hi)
        if r == 1:
            return _merge_top_1(A[0], B[0])
        if len(A) == 1:
            return _merge_full_1(A[0], B[0])
        return _merge_top_2(A, B)

    return build(0, n_leaves)


def _make_topk_sc(n, k, kpad, bt):
    m = n // 16
    nv = m // 16
    tw = _T // _NW
    nb = tw // bt
    r = 2 if k > 16 else 1

    mesh = plsc.VectorSubcoreMesh(core_axis_name="c", subcore_axis_name="s")

    @functools.partial(
        pl.kernel,
        mesh=mesh,
        out_type=[
            jax.ShapeDtypeStruct((_T * kpad,), jnp.float32),
            jax.ShapeDtypeStruct((_T * kpad,), jnp.int32),
        ],
        scratch_types=[
            pltpu.VMEM((bt, n), jnp.float32),
            pltpu.VMEM((bt, m), jnp.float32),
            pltpu.VMEM((bt * k + 16,), jnp.int32),
            pltpu.VMEM((bt * kpad,), jnp.float32),
            pltpu.VMEM((bt * kpad,), jnp.int32),
        ],
        compiler_params=pltpu.CompilerParams(needs_layout_passes=False),
    )
    def topk_sc(s_hbm, mx_hbm, ov_hbm, oi_hbm,
                rows_ref, mx_ref, cand_ref, outv_ref, outi_ref):
        wid = lax.axis_index("s") * _NSC + lax.axis_index("c")
        wbase = wid * tw
        ii = lax.iota(jnp.int32, 16)

        def batch_body(b, _):
            base = wbase + b * bt
            pltpu.sync_copy(mx_hbm.at[pl.ds(base, bt)], mx_ref)
            pltpu.sync_copy(s_hbm.at[pl.ds(base, bt)], rows_ref)

            def tok_body(t, _):
                tvec = jnp.full((16,), t, jnp.int32)

                def load_mx(j):
                    v = -plsc.load_gather(mx_ref, [tvec, j * 16 + ii])
                    return v, v

                node = _tree_topk(load_mx, nv, r)
                tnode = node[1][0] if k > 16 else node[0][0]
                tval = -tnode[(k - 1) % 16]

                def scan_body(j, prev):
                    v = plsc.load_gather(mx_ref, [tvec, j * 16 + ii])
                    q = v >= tval
                    qi = q.astype(jnp.int32)
                    cs = plsc.cumsum(qi)
                    pos = prev + cs - 1
                    wm = q & (pos < k)
                    plsc.store_scatter(cand_ref, [t * k + pos],
                                       j * 16 + ii, mask=wm)
                    return prev + jnp.sum(qi)

                lax.fori_loop(0, nv, scan_body, jnp.int32(0))

                cvecs = [cand_ref[pl.ds(t * k + 16 * c, 16)]
                         for c in range((k + 15) // 16)]

                def load_cand(j):
                    cid = cvecs[j // 16][j % 16]
                    eidx = cid * 16 + ii
                    v = plsc.load_gather(rows_ref, [tvec, eidx])
                    return -v, eidx

                node = _tree_topk(load_cand, k, r)
                obase = t * kpad
                outv_ref[pl.ds(obase, 16)] = -node[0][0]
                outi_ref[pl.ds(obase, 16)] = node[0][1]
                if k > 16:
                    outv_ref[pl.ds(obase + 16, 16)] = -node[1][0]
                    outi_ref[pl.ds(obase + 16, 16)] = node[1][1]
                return 0

            lax.fori_loop(0, bt, tok_body, 0)

            pltpu.sync_copy(outv_ref, ov_hbm.at[pl.ds(base * kpad, bt * kpad)])
            pltpu.sync_copy(outi_ref, oi_hbm.at[pl.ds(base * kpad, bt * kpad)])
            return 0

        lax.fori_loop(0, nb, batch_body, 0)

    return topk_sc


def _topk_sc(scores, maxima, n, k):
    kpad = k if k % 16 == 0 else 16
    bt = 16 if n <= 4096 else 8
    ov, oi = _make_topk_sc(n, k, kpad, bt)(scores, maxima)
    tv = ov.reshape(_T, kpad)[:, :k]
    ti = oi.reshape(_T, kpad)[:, :k]
    return tv, ti


def _gate_body(s_ref, tau_ref, g_ref):
    s = s_ref[...]
    tau = tau_ref[...]
    raw = s - tau
    gate = jnp.where(raw > 0, raw, 1e-08 * jnp.exp(raw))
    eg = jnp.exp(gate) - 1.0
    gsum = eg.sum(axis=-1, keepdims=True) + 1e-08
    gstr = jnp.tanh(eg.max(axis=-1, keepdims=True))
    g_ref[...] = eg / gsum * gstr


def _gate(topk_scores, tau, k):
    blk = 512
    return pl.pallas_call(
        _gate_body,
        grid=(_T // blk,),
        in_specs=[
            pl.BlockSpec((blk, k), lambda i: (i, 0)),
            pl.BlockSpec((blk, 1), lambda i: (i, 0)),
        ],
        out_specs=pl.BlockSpec((blk, k), lambda i: (i, 0)),
        out_shape=jax.ShapeDtypeStruct((_T, k), jnp.float32),
    )(topk_scores, tau)


def kernel(x, qk_emb, v_emb, know_emb, W_attn, b_attn, W_tau_attn, b_tau_attn,
           W_know, b_know, W_tau_know, b_tau_know):
    x2d = x.reshape(_T, _DM)
    qk_n = _normalize(qk_emb)
    v_n = _normalize(v_emb)
    know_n = _normalize(know_emb)

    W4 = jnp.concatenate([W_tau_attn, W_tau_know], axis=1)
    W4 = jnp.pad(W4, ((0, 0), (0, 4)))
    b4 = jnp.pad(jnp.concatenate([b_tau_attn, b_tau_know]), (0, 4)).reshape(1, 8)
    tau_all = _tau(x2d, W4, b4)

    def route(wslice, bslice, emb_n, n, tb, k, tau_col):
        s, m, aux = _scores(x2d, wslice, bslice, emb_n, n, tb)
        tv, ti = _topk_sc(s, m, n, k)
        g = _gate(tv, tau_all[:, tau_col:tau_col + 1], k)
        return g.reshape(_B, _S, k), ti.reshape(_B, _S, k), aux[0, 0]

    g_Q, i_Q, aux_q = route(W_attn[:, 0:_DB], b_attn[0:_DB].reshape(1, _DB),
                            qk_n, _NQK, 128, _KQK, 0)
    g_K, i_K, _ = route(W_attn[:, _DB:2 * _DB], b_attn[_DB:2 * _DB].reshape(1, _DB),
                        qk_n, _NQK, 128, _KQK, 1)
    g_V, i_V, aux_v = route(W_attn[:, 2 * _DB:], b_attn[2 * _DB:].reshape(1, _DB),
                            v_n, _NV, 128, _KV, 2)
    g_know, i_know, aux_k = route(W_know, b_know.reshape(1, _DB),
                                  know_n, _NKNOW, 64, _KKNOW, 3)

    aux_attn = aux_q * _NQK * 3 + aux_v * _NV
    aux_know = aux_k * _NKNOW
    return (g_Q, i_Q, g_K, i_K, g_V, i_V, aux_attn, g_know, i_know, aux_know)
